# baseline (speedup 1.0000x reference)
import functools

import jax
import jax.numpy as jnp
from jax import lax
from jax.experimental import pallas as pl
from jax.experimental.pallas import tpu as pltpu
from jax.experimental.pallas import tpu_sc as plsc

B = 4096
IN = 128
H = 128
R = 8
MPHY = 4
K = 1024
D = 256
TAU = 1.0
BPS = 2
KBITS = 10
TEMP_MOD = 0.5

TB = 256
NBLK = B // TB


def _router_vq_kernel(phi_ref, z_ref, ln_g_ref, ln_b_ref, W1_ref, b1_ref,
                      W2_ref, b2_ref, We_ref, be_ref, Wm_ref, bm_ref,
                      cb_ref, T_ref,
                      jp_ref, mi_ref, gate_ref, idx_ref, xs_ref):
    phi = phi_ref[...]
    mu = jnp.mean(phi, axis=-1, keepdims=True)
    var = jnp.mean((phi - mu) ** 2, axis=-1, keepdims=True)
    phin = (phi - mu) / jnp.sqrt(var + 1e-5) * ln_g_ref[...] + ln_b_ref[...]
    h = jax.nn.gelu(jnp.dot(phin, W1_ref[...]) + b1_ref[...])
    h = jax.nn.gelu(jnp.dot(h, W2_ref[...]) + b2_ref[...])
    logits_e = jnp.dot(h, We_ref[...]) + be_ref[...]
    logits_m = jnp.dot(h, Wm_ref[...]) + bm_ref[...]
    p_e = jax.nn.softmax(logits_e / TAU, axis=-1)
    p_m = jax.nn.softmax(logits_m / TAU, axis=-1)
    jl = (logits_e[:, :, None] + logits_m[:, None, :]).reshape(TB, R * MPHY)
    jp = (p_e[:, :, None] * p_m[:, None, :]).reshape(TB, R * MPHY)
    iota_j = lax.broadcasted_iota(jnp.int32, (TB, R * MPHY), 1)
    jl_max = jnp.max(jl, axis=-1, keepdims=True)
    mi = jnp.min(jnp.where(jl == jl_max, iota_j, R * MPHY), axis=-1,
                 keepdims=True)
    gate = jnp.sum(jnp.where(iota_j == mi, jp, 0.0), axis=-1, keepdims=True)
    expert = mi // MPHY

    jp_ref[...] = jp
    mi_ref[...] = mi
    gate_ref[...] = gate

    z = z_ref[...]
    rn = jnp.sum(z * z, axis=-1, keepdims=True)
    iota_k = lax.broadcasted_iota(jnp.int32, (TB, K), 1)

    def body(e, bidx):
        cbe = cb_ref[pl.ds(e * K, K), :]
        mm = lax.dot_general(z, cbe, (((1,), (1,)), ((), ())))
        nce = jnp.sum(cbe * cbe, axis=-1)
        d = (rn - 2.0 * mm) + nce[None, :]
        mv = jnp.min(d, axis=-1, keepdims=True)
        ii = jnp.min(jnp.where(d == mv, iota_k, R * K), axis=-1,
                     keepdims=True) + e * K
        return jnp.where(expert == e, ii, bidx)

    bidx = lax.fori_loop(0, R, body, jnp.zeros((TB, 1), jnp.int32))
    idx_ref[...] = bidx

    code = jnp.bitwise_and(bidx, K - 1)
    cols = []
    for s in range(KBITS // BPS):
        pr = jnp.bitwise_and(
            lax.shift_right_logical(code, KBITS - BPS - BPS * s), 3)
        for c in range(2):
            v = jnp.where(pr == 0, T_ref[0, c],
                jnp.where(pr == 1, T_ref[1, c],
                jnp.where(pr == 2, T_ref[2, c], T_ref[3, c])))
            cols.append(v)
    xs_ref[...] = jnp.concatenate(cols, axis=1)


def _router_vq(phi, z, ln_g, ln_b, W1, b1, W2, b2, We, be, Wm, bm, cb_flat, T):
    const_spec = lambda shape: pl.BlockSpec(shape, lambda i: (0, 0))
    tok_spec = lambda shape: pl.BlockSpec(shape, lambda i: (i, 0))
    return pl.pallas_call(
        _router_vq_kernel,
        grid=(NBLK,),
        in_specs=[
            tok_spec((TB, IN)),
            tok_spec((TB, D)),
            const_spec((1, IN)),
            const_spec((1, IN)),
            const_spec((IN, H)),
            const_spec((1, H)),
            const_spec((H, H)),
            const_spec((1, H)),
            const_spec((H, R)),
            const_spec((1, R)),
            const_spec((H, MPHY)),
            const_spec((1, MPHY)),
            const_spec((R * K, D)),
            const_spec((4, 2)),
        ],
        out_specs=[
            tok_spec((TB, R * MPHY)),
            tok_spec((TB, 1)),
            tok_spec((TB, 1)),
            tok_spec((TB, 1)),
            tok_spec((TB, 2 * (KBITS // BPS))),
        ],
        out_shape=[
            jax.ShapeDtypeStruct((B, R * MPHY), jnp.float32),
            jax.ShapeDtypeStruct((B, 1), jnp.int32),
            jax.ShapeDtypeStruct((B, 1), jnp.float32),
            jax.ShapeDtypeStruct((B, 1), jnp.int32),
            jax.ShapeDtypeStruct((B, 2 * (KBITS // BPS)), jnp.float32),
        ],
    )(phi, z, ln_g.reshape(1, IN), ln_b.reshape(1, IN), W1, b1.reshape(1, H),
      W2, b2.reshape(1, H), We, be.reshape(1, R), Wm, bm.reshape(1, MPHY),
      cb_flat, T)


def _sc_gather(cb_flat, idx_flat):
    info = plsc.get_sparse_core_info()
    nc, ns = info.num_cores, info.num_subcores
    nw = nc * ns
    bpw = B // nw
    mesh = plsc.VectorSubcoreMesh(core_axis_name="c", subcore_axis_name="s")

    @functools.partial(
        pl.kernel,
        out_type=jax.ShapeDtypeStruct((B, D), jnp.float32),
        mesh=mesh,
        scratch_types=[
            pltpu.VMEM((bpw,), jnp.int32),
            pltpu.VMEM((bpw, D), jnp.float32),
            pltpu.SemaphoreType.DMA,
        ],
    )
    def gather_k(cb_hbm, idx_hbm, out_hbm, idx_v, rows_v, sem):
        wid = lax.axis_index("s") * nc + lax.axis_index("c")
        base = wid * bpw
        pltpu.sync_copy(idx_hbm.at[pl.ds(base, bpw)], idx_v)
        pltpu.async_copy(cb_hbm.at[idx_v], rows_v, sem).wait()
        pltpu.sync_copy(rows_v, out_hbm.at[pl.ds(base, bpw)])

    return gather_k(cb_flat, idx_flat)


def _combine_kernel(z_ref, zq_ref, gate_ref, out_ref):
    z = z_ref[...]
    out_ref[...] = (z + (zq_ref[...] - z)) * gate_ref[...]


def _combine(z, zq, gate):
    tok_spec = pl.BlockSpec((TB, D), lambda i: (i, 0))
    return pl.pallas_call(
        _combine_kernel,
        grid=(NBLK,),
        in_specs=[tok_spec, tok_spec, pl.BlockSpec((TB, 1), lambda i: (i, 0))],
        out_specs=tok_spec,
        out_shape=jax.ShapeDtypeStruct((B, D), jnp.float32),
    )(z, zq, gate)


def _int_to_bits(x, num_bits):
    shifts = jnp.arange(num_bits - 1, -1, -1)
    return ((x[..., None] >> shifts) & 1).astype(jnp.float32)


def _qam_table():
    import numpy as np
    m_side = int(np.sqrt(1 << BPS))
    levels = jnp.arange(-(m_side - 1), m_side + 1, 2).astype(jnp.float32)
    xs, ys = jnp.meshgrid(levels, levels, indexing='ij')
    pts = jnp.stack([xs.reshape(-1), ys.reshape(-1)], axis=-1)
    max_power = (pts ** 2).sum(axis=-1).max()
    const = pts / jnp.sqrt(max_power + 1e-9)
    cand_bits = _int_to_bits(jnp.arange(1 << BPS), BPS)
    patt = cand_bits
    d_bits = ((patt[:, None, :] - cand_bits[None, :, :]) ** 2).sum(axis=-1)
    w_sym = jax.nn.softmax(-d_bits / max(TEMP_MOD, 1e-6), axis=1)
    return w_sym @ const


def kernel(z, phi, ln_g, ln_b, W1, b1, W2, b2, We, be, Wm, bm, codebooks):
    cb_flat = codebooks.reshape(R * K, D)
    T = _qam_table()
    jp, mi, gate, idx, xs = _router_vq(phi, z, ln_g, ln_b, W1, b1, W2, b2,
                                       We, be, Wm, bm, cb_flat, T)
    zq = _sc_gather(cb_flat, idx.reshape(B))
    out = _combine(z, zq, gate)
    x_sym = xs.reshape(B, KBITS // BPS, 2)
    return (out, x_sym, jp, mi.reshape(B))
